# Initial kernel scaffold; baseline (speedup 1.0000x reference)
import functools

import jax
import jax.numpy as jnp
from jax import lax
from jax.experimental import pallas as pl
from jax.experimental.pallas import tpu as pltpu
from jax.experimental.pallas import tpu_sc as plsc

VOCAB = 100000
D = 128
BATCH = 4096
SEQ = 200
N = BATCH * SEQ
NC, NS = 2, 16
NW = NC * NS
PER_W = N // NW
G = 128
K = 2
C = K * G
CHUNKS = PER_W // C
LANES = 16
DB = D // LANES


def _sc_body(seq_hbm, lab_hbm, tok_hbm, seg_hbm, out_hbm,
             idx_v, lab_v, rows_v, seg_v, gsem):
    wid = lax.axis_index("s") * NC + lax.axis_index("c")
    base = wid * PER_W

    pltpu.sync_copy(seg_hbm, seg_v)
    seg_regs = [[seg_v[k, pl.ds(db * LANES, LANES)] for db in range(DB)]
                for k in range(3)]

    def chunk_body(c, _):
        off = base + c * C
        pltpu.sync_copy(seq_hbm.at[pl.ds(off // G, K)], idx_v)
        pltpu.sync_copy(lab_hbm.at[pl.ds(off, C)], lab_v)
        copies = [pltpu.async_copy(tok_hbm.at[idx_v.at[j]],
                                   rows_v.at[pl.ds(j * G, G)], gsem)
                  for j in range(K)]
        for cp in copies:
            cp.wait()

        def tok_body(t, _):
            lab = plsc.load_gather(lab_v, [jnp.full((LANES,), t, jnp.int32)])
            m1 = lab == 1
            m2 = lab == 2
            for db in range(DB):
                sl = pl.ds(db * LANES, LANES)
                sv = jnp.where(m1, seg_regs[1][db], seg_regs[0][db])
                sv = jnp.where(m2, seg_regs[2][db], sv)
                rows_v[t, sl] = rows_v[t, sl] + sv
            return 0

        lax.fori_loop(0, C, tok_body, 0)
        pltpu.sync_copy(rows_v, out_hbm.at[pl.ds(off, C)])
        return 0

    lax.fori_loop(0, CHUNKS, chunk_body, 0)


@jax.jit
def _embed(seq2d, lab, token_table, segment_table):
    fn = functools.partial(
        pl.kernel,
        out_type=jax.ShapeDtypeStruct((N, D), jnp.float32),
        mesh=plsc.VectorSubcoreMesh(core_axis_name="c", subcore_axis_name="s"),
        scratch_types=[
            pltpu.VMEM((K, G), jnp.int32),
            pltpu.VMEM((C,), jnp.int32),
            pltpu.VMEM((C, D), jnp.float32),
            pltpu.VMEM((3, D), jnp.float32),
            pltpu.SemaphoreType.DMA,
        ],
    )(_sc_body)
    return fn(seq2d, lab, token_table, segment_table)


def kernel(sequence, segment_label, token_table, segment_table):
    seq2d = sequence.reshape(N // G, G)
    lab = segment_label.reshape(N)
    out = _embed(seq2d, lab, token_table, segment_table)
    return out.reshape(BATCH, SEQ, D)


# speedup vs baseline: 8.1760x; 8.1760x over previous
import functools

import jax
import jax.numpy as jnp
from jax import lax
from jax.experimental import pallas as pl
from jax.experimental.pallas import tpu as pltpu
from jax.experimental.pallas import tpu_sc as plsc

VOCAB = 100000
D = 128
BATCH = 4096
SEQ = 200
N = BATCH * SEQ
NC, NS = 2, 16
NW = NC * NS
PER_W = N // NW
G = 128
K = 2
C = K * G
IDXROWS = 8
SUP = IDXROWS * G
SUBS = SUP // C
SUPS = PER_W // SUP
LANES = 16
DB = D // LANES


def _sc_body(seq_hbm, lab_hbm, tok_hbm, seg_hbm, out_hbm,
             idx_v, lab_v, rows_v, seg_v, gsem):
    wid = lax.axis_index("s") * NC + lax.axis_index("c")
    base = wid * PER_W

    pltpu.sync_copy(seg_hbm, seg_v)
    s0 = [seg_v[0, pl.ds(db * LANES, LANES)] for db in range(DB)]
    d1 = [seg_v[1, pl.ds(db * LANES, LANES)] - s0[db] for db in range(DB)]
    d2 = [seg_v[2, pl.ds(db * LANES, LANES)] - seg_v[1, pl.ds(db * LANES, LANES)]
          for db in range(DB)]

    def sup_body(s, _):
        soff = base + s * SUP
        pltpu.sync_copy(seq_hbm.at[pl.ds(pl.multiple_of(soff // G, 8), IDXROWS)],
                        idx_v)
        pltpu.sync_copy(lab_hbm.at[pl.ds(pl.multiple_of(soff, SUP), SUP)], lab_v)

        for sub in range(SUBS):
            off = soff + sub * C
            copies = [pltpu.async_copy(tok_hbm.at[idx_v.at[K * sub + j]],
                                       rows_v.at[pl.ds(j * G, G)], gsem)
                      for j in range(K)]
            for cp in copies:
                cp.wait()

            def grp_body(g, _, sub=sub):
                lab16 = lab_v[pl.ds(sub * C + g * LANES, LANES)]
                for i in range(LANES):
                    labi = lab16[i]
                    a1 = jnp.full((LANES,),
                                  jnp.minimum(labi, 1), jnp.int32).astype(jnp.float32)
                    a2 = jnp.full((LANES,),
                                  jnp.maximum(labi - 1, 0), jnp.int32).astype(jnp.float32)
                    t = g * LANES + i
                    for db in range(DB):
                        sl = pl.ds(db * LANES, LANES)
                        sv = s0[db] + a1 * d1[db] + a2 * d2[db]
                        rows_v[t, sl] = rows_v[t, sl] + sv
                return 0

            lax.fori_loop(0, C // LANES, grp_body, 0)
            pltpu.sync_copy(rows_v, out_hbm.at[pl.ds(pl.multiple_of(off, C), C)])
        return 0

    lax.fori_loop(0, SUPS, sup_body, 0)


@jax.jit
def _embed(seq2d, lab, token_table, segment_table):
    fn = functools.partial(
        pl.kernel,
        out_type=jax.ShapeDtypeStruct((N, D), jnp.float32),
        mesh=plsc.VectorSubcoreMesh(core_axis_name="c", subcore_axis_name="s"),
        scratch_types=[
            pltpu.VMEM((IDXROWS, G), jnp.int32),
            pltpu.VMEM((SUP,), jnp.int32),
            pltpu.VMEM((C, D), jnp.float32),
            pltpu.VMEM((3, D), jnp.float32),
            pltpu.SemaphoreType.DMA,
        ],
    )(_sc_body)
    return fn(seq2d, lab, token_table, segment_table)


def kernel(sequence, segment_label, token_table, segment_table):
    seq2d = sequence.reshape(N // G, G)
    lab = segment_label.reshape(N)
    out = _embed(seq2d, lab, token_table, segment_table)
    return out.reshape(BATCH, SEQ, D)


# speedup vs baseline: 10.5207x; 1.2868x over previous
import functools

import jax
import jax.numpy as jnp
from jax import lax
from jax.experimental import pallas as pl
from jax.experimental.pallas import tpu as pltpu
from jax.experimental.pallas import tpu_sc as plsc

VOCAB = 100000
D = 128
BATCH = 4096
SEQ = 200
N = BATCH * SEQ
NC, NS = 2, 16
NW = NC * NS
PER_W = N // NW
G = 128
STEPS = PER_W // G
NBUF = 4
BODIES = STEPS // NBUF
IDXROWS = PER_W // G
LANES = 16
DB = D // LANES


def _sc_body(seq_hbm, lab_hbm, tok_hbm, seg_hbm, out_hbm,
             idx_v, lab_v, rows0, rows1, rows2, rows3, seg_v,
             gsem0, gsem1, gsem2, gsem3, ssem0, ssem1, ssem2, ssem3):
    wid = lax.axis_index("s") * NC + lax.axis_index("c")
    base = wid * PER_W
    rows = (rows0, rows1, rows2, rows3)
    gsems = (gsem0, gsem1, gsem2, gsem3)
    ssems = (ssem0, ssem1, ssem2, ssem3)

    pltpu.sync_copy(seq_hbm.at[pl.ds(pl.multiple_of(base // G, 8), IDXROWS)],
                    idx_v)
    pltpu.sync_copy(lab_hbm.at[pl.ds(pl.multiple_of(base, PER_W), PER_W)],
                    lab_v)
    pltpu.sync_copy(seg_hbm, seg_v)
    s0 = [seg_v[0, pl.ds(db * LANES, LANES)] for db in range(DB)]
    d1 = [seg_v[1, pl.ds(db * LANES, LANES)] - s0[db] for db in range(DB)]
    d2 = [seg_v[2, pl.ds(db * LANES, LANES)] - seg_v[1, pl.ds(db * LANES, LANES)]
          for db in range(DB)]

    def compute(step, b):
        def grp_body(g, _):
            lab16 = lab_v[pl.ds(step * G + g * LANES, LANES)]
            for i in range(LANES):
                labi = lab16[i]
                a1 = jnp.full((LANES,),
                              jnp.minimum(labi, 1), jnp.int32).astype(jnp.float32)
                a2 = jnp.full((LANES,),
                              jnp.maximum(labi - 1, 0), jnp.int32).astype(jnp.float32)
                t = g * LANES + i
                for db in range(DB):
                    sl = pl.ds(db * LANES, LANES)
                    sv = s0[db] + a1 * d1[db] + a2 * d2[db]
                    rows[b][t, sl] = rows[b][t, sl] + sv
            return 0
        lax.fori_loop(0, G // LANES, grp_body, 0)

    def loop_body(i, _):
        s0i = i * NBUF
        gcp = [pltpu.async_copy(tok_hbm.at[idx_v.at[s0i + b]], rows[b],
                                gsems[b])
               for b in range(NBUF)]
        scp = []
        for b in range(NBUF):
            gcp[b].wait()
            compute(s0i + b, b)
            off = base + (s0i + b) * G
            scp.append(pltpu.async_copy(
                rows[b], out_hbm.at[pl.ds(pl.multiple_of(off, G), G)],
                ssems[b]))
        for cp in scp:
            cp.wait()
        return 0

    lax.fori_loop(0, BODIES, loop_body, 0)


@jax.jit
def _embed(seq2d, lab, token_table, segment_table):
    fn = functools.partial(
        pl.kernel,
        out_type=jax.ShapeDtypeStruct((N, D), jnp.float32),
        mesh=plsc.VectorSubcoreMesh(core_axis_name="c", subcore_axis_name="s"),
        scratch_types=[
            pltpu.VMEM((IDXROWS, G), jnp.int32),
            pltpu.VMEM((PER_W,), jnp.int32),
            pltpu.VMEM((G, D), jnp.float32),
            pltpu.VMEM((G, D), jnp.float32),
            pltpu.VMEM((G, D), jnp.float32),
            pltpu.VMEM((G, D), jnp.float32),
            pltpu.VMEM((3, D), jnp.float32),
            pltpu.SemaphoreType.DMA,
            pltpu.SemaphoreType.DMA,
            pltpu.SemaphoreType.DMA,
            pltpu.SemaphoreType.DMA,
            pltpu.SemaphoreType.DMA,
            pltpu.SemaphoreType.DMA,
            pltpu.SemaphoreType.DMA,
            pltpu.SemaphoreType.DMA,
        ],
    )(_sc_body)
    return fn(seq2d, lab, token_table, segment_table)


def kernel(sequence, segment_label, token_table, segment_table):
    seq2d = sequence.reshape(N // G, G)
    lab = segment_label.reshape(N)
    out = _embed(seq2d, lab, token_table, segment_table)
    return out.reshape(BATCH, SEQ, D)
